# baseline (speedup 1.0000x reference)
import functools

import jax
import jax.numpy as jnp
from jax import lax
from jax.experimental import pallas as pl
from jax.experimental.pallas import tpu as pltpu
from jax.experimental.pallas import tpu_sc as plsc

_NUM_SELECT = 60
_NBP = 13
_KPAD = 64
_NW = 32
_KPW = _NUM_SELECT * 16 // _NW
_L = 16


def _topk_kernel(x_ref, scores_ref, labels_ref, rows_ref, xs_ref, *, num_classes):
    B, F = x_ref.shape
    xs_ref[...] = x_ref[...]
    col = jax.lax.broadcasted_iota(jnp.int32, (B, F), 1)
    lane = jax.lax.broadcasted_iota(jnp.int32, (B, _KPAD), 1)
    neg = jnp.float32(-jnp.inf)

    def body(i, carry):
        vals, idxs = carry
        x = xs_ref[...]
        m = jnp.max(x, axis=1, keepdims=True)
        loc = jnp.min(jnp.where(x >= m, col, F), axis=1, keepdims=True)
        xs_ref[...] = jnp.where(col == loc, neg, x)
        sel = lane == i
        vals = jnp.where(sel, m, vals)
        idxs = jnp.where(sel, loc, idxs)
        return vals, idxs

    vals = jnp.full((B, _KPAD), neg, jnp.float32)
    idxs = jnp.zeros((B, _KPAD), jnp.int32)
    vals, idxs = jax.lax.fori_loop(0, _NUM_SELECT, body, (vals, idxs))
    scores_ref[...] = jax.nn.sigmoid(vals)
    labels_ref[...] = idxs % num_classes
    rows_ref[...] = idxs // num_classes


def _sc_gather_kernel(rows_hbm, ts_hbm, kp_hbm, out_hbm,
                      rows_v, row_buf, tsv, out_v, sem, *, d):
    cid = lax.axis_index("c")
    sid = lax.axis_index("s")
    wid = sid * 2 + cid
    b = sid

    pltpu.sync_copy(rows_hbm.at[wid], rows_v)
    pltpu.sync_copy(ts_hbm, tsv)

    rv0 = rows_v[pl.ds(0, _L)]
    rv1 = rows_v[pl.ds(_L, _L)]
    copies = []
    for s in range(_KPW):
        r = rv0[s] if s < _L else rv1[s - _L]
        c = pltpu.async_copy(kp_hbm.at[b, r], row_buf.at[s], sem)
        copies.append(c)
    for c in copies:
        c.wait()

    lane = lax.iota(jnp.int32, _L)
    tsb_v = jnp.broadcast_to(2 * b, (_L,))
    sv = plsc.load_gather(tsv, [tsb_v + (lane & 1)])
    for s in range(_KPW):
        va = row_buf[s, pl.ds(0, _L)] * sv
        vb = row_buf[s, pl.ds(d - _L, _L)] * sv
        out_v[pl.ds(s * d, _L)] = va
        out_v[pl.ds(s * d + d - _L, _L)] = vb

    pltpu.sync_copy(out_v, out_hbm.at[wid])


def kernel(pred_logits, pred_keypoints, target_sizes):
    B, N, C = pred_logits.shape
    D = pred_keypoints.shape[-1]
    flat = pred_logits.reshape(B, N * C)

    scores64, labels64, rows64 = pl.pallas_call(
        functools.partial(_topk_kernel, num_classes=C),
        out_shape=[
            jax.ShapeDtypeStruct((B, _KPAD), jnp.float32),
            jax.ShapeDtypeStruct((B, _KPAD), jnp.int32),
            jax.ShapeDtypeStruct((B, _KPAD), jnp.int32),
        ],
        scratch_shapes=[pltpu.VMEM((B, N * C), jnp.float32)],
    )(flat)

    rows32 = jnp.pad(
        rows64[:, :_NUM_SELECT].reshape(_NW, _KPW), ((0, 0), (0, 2))
    )
    ts_flat = target_sizes.reshape(B * 2)
    out_pad = ((_KPW * D + _L) // _L) * _L

    mesh = plsc.VectorSubcoreMesh(core_axis_name="c", subcore_axis_name="s")
    sc_gather = functools.partial(
        pl.kernel,
        mesh=mesh,
        compiler_params=pltpu.CompilerParams(needs_layout_passes=False),
        out_type=jax.ShapeDtypeStruct((_NW, out_pad), jnp.float32),
        scratch_types=[
            pltpu.VMEM((_KPW + 2,), jnp.int32),
            pltpu.VMEM((_KPW, D), jnp.float32),
            pltpu.VMEM((B * 2,), jnp.float32),
            pltpu.VMEM((out_pad,), jnp.float32),
            pltpu.SemaphoreType.DMA,
        ],
    )(functools.partial(_sc_gather_kernel, d=D))
    out32 = sc_gather(rows32, ts_flat, pred_keypoints)

    scores = scores64[:, :_NUM_SELECT]
    labels = labels64[:, :_NUM_SELECT]
    kp26 = out32[:, : _KPW * D].reshape(B, _NUM_SELECT, D)
    kpr = kp26.reshape(B, _NUM_SELECT, _NBP, 2)
    kpr = jnp.concatenate([kpr, jnp.ones_like(kpr[..., :1])], axis=-1)
    return scores, labels, kpr.reshape(B, _NUM_SELECT, _NBP * 3)


# speedup vs baseline: 1.1576x; 1.1576x over previous
import functools

import jax
import jax.numpy as jnp
from jax.experimental import pallas as pl
from jax.experimental.pallas import tpu as pltpu

_NUM_SELECT = 60
_NBP = 13
_KPAD = 64
_PER_PASS = 2


def _topk_kernel(x_ref, scores_ref, labels_ref, rows_ref, xs_ref, *, num_classes):
    B, F = x_ref.shape
    xs_ref[...] = x_ref[...]
    col = jax.lax.broadcasted_iota(jnp.int32, (B, F), 1)
    lane = jax.lax.broadcasted_iota(jnp.int32, (B, _KPAD), 1)
    neg = jnp.float32(-jnp.inf)

    def body(i, carry):
        vals, idxs = carry
        x = xs_ref[...]
        for j in range(_PER_PASS):
            m = jnp.max(x, axis=1, keepdims=True)
            loc = jnp.min(jnp.where(x >= m, col, F), axis=1, keepdims=True)
            x = jnp.where(col == loc, neg, x)
            sel = lane == i * _PER_PASS + j
            vals = jnp.where(sel, m, vals)
            idxs = jnp.where(sel, loc, idxs)
        xs_ref[...] = x
        return vals, idxs

    vals = jnp.full((B, _KPAD), neg, jnp.float32)
    idxs = jnp.zeros((B, _KPAD), jnp.int32)
    vals, idxs = jax.lax.fori_loop(0, _NUM_SELECT // _PER_PASS, body, (vals, idxs))
    scores_ref[...] = jax.nn.sigmoid(vals)
    labels_ref[...] = idxs % num_classes
    rows_ref[...] = idxs // num_classes


def _gather_kernel(rows_ref, ts_ref, kp_ref, out_ref, scratch, sem):
    B, NS, D = out_ref.shape
    copies = []
    for b in range(B):
        for s in range(NS):
            c = pltpu.make_async_copy(
                kp_ref.at[b, rows_ref[b, s]], scratch.at[b, s], sem
            )
            c.start()
            copies.append(c)
    for c in copies:
        c.wait()
    lane = jax.lax.broadcasted_iota(jnp.int32, (NS, D), 1)
    even = lane % 2 == 0
    for b in range(B):
        sx = ts_ref[b, 0]
        sy = ts_ref[b, 1]
        out_ref[b] = scratch[b] * jnp.where(even, sx, sy)


def kernel(pred_logits, pred_keypoints, target_sizes):
    B, N, C = pred_logits.shape
    D = pred_keypoints.shape[-1]
    flat = pred_logits.reshape(B, N * C)

    scores64, labels64, rows64 = pl.pallas_call(
        functools.partial(_topk_kernel, num_classes=C),
        out_shape=[
            jax.ShapeDtypeStruct((B, _KPAD), jnp.float32),
            jax.ShapeDtypeStruct((B, _KPAD), jnp.int32),
            jax.ShapeDtypeStruct((B, _KPAD), jnp.int32),
        ],
        scratch_shapes=[pltpu.VMEM((B, N * C), jnp.float32)],
    )(flat)

    rows = rows64[:, :_NUM_SELECT]
    kp26 = pl.pallas_call(
        _gather_kernel,
        in_specs=[
            pl.BlockSpec(memory_space=pltpu.SMEM),
            pl.BlockSpec(memory_space=pltpu.SMEM),
            pl.BlockSpec(memory_space=pl.ANY),
        ],
        out_shape=jax.ShapeDtypeStruct((B, _NUM_SELECT, D), jnp.float32),
        scratch_shapes=[
            pltpu.VMEM((B, _NUM_SELECT, D), jnp.float32),
            pltpu.SemaphoreType.DMA,
        ],
    )(rows, target_sizes, pred_keypoints)

    scores = scores64[:, :_NUM_SELECT]
    labels = labels64[:, :_NUM_SELECT]
    kpr = kp26.reshape(B, _NUM_SELECT, _NBP, 2)
    kpr = jnp.concatenate([kpr, jnp.ones_like(kpr[..., :1])], axis=-1)
    return scores, labels, kpr.reshape(B, _NUM_SELECT, _NBP * 3)


# speedup vs baseline: 1.1781x; 1.0177x over previous
import functools

import jax
import jax.numpy as jnp
from jax.experimental import pallas as pl
from jax.experimental.pallas import tpu as pltpu

_NUM_SELECT = 60
_NBP = 13
_KPAD = 64
_PER_PASS = 4


def _topk_kernel(x_ref, scores_ref, labels_ref, rows_ref, xs_ref, *, num_classes):
    B, F = x_ref.shape
    xs_ref[...] = x_ref[...]
    col = jax.lax.broadcasted_iota(jnp.int32, (B, F), 1)
    lane = jax.lax.broadcasted_iota(jnp.int32, (B, _KPAD), 1)
    neg = jnp.float32(-jnp.inf)

    def body(i, carry):
        vals, idxs = carry
        x = xs_ref[...]
        for j in range(_PER_PASS):
            m = jnp.max(x, axis=1, keepdims=True)
            loc = jnp.min(jnp.where(x >= m, col, F), axis=1, keepdims=True)
            x = jnp.where(col == loc, neg, x)
            sel = lane == i * _PER_PASS + j
            vals = jnp.where(sel, m, vals)
            idxs = jnp.where(sel, loc, idxs)
        xs_ref[...] = x
        return vals, idxs

    vals = jnp.full((B, _KPAD), neg, jnp.float32)
    idxs = jnp.zeros((B, _KPAD), jnp.int32)
    vals, idxs = jax.lax.fori_loop(0, _NUM_SELECT // _PER_PASS, body, (vals, idxs))
    scores_ref[...] = jax.nn.sigmoid(vals)
    labels_ref[...] = idxs % num_classes
    rows_ref[...] = idxs // num_classes


def _gather_kernel(rows_ref, ts_ref, kp_ref, out_ref, scratch, sem):
    B, NS, D = out_ref.shape
    copies = []
    for b in range(B):
        for s in range(NS):
            c = pltpu.make_async_copy(
                kp_ref.at[b, rows_ref[b, s]], scratch.at[b, s], sem
            )
            c.start()
            copies.append(c)
    for c in copies:
        c.wait()
    lane = jax.lax.broadcasted_iota(jnp.int32, (NS, D), 1)
    even = lane % 2 == 0
    for b in range(B):
        sx = ts_ref[b, 0]
        sy = ts_ref[b, 1]
        out_ref[b] = scratch[b] * jnp.where(even, sx, sy)


def kernel(pred_logits, pred_keypoints, target_sizes):
    B, N, C = pred_logits.shape
    D = pred_keypoints.shape[-1]
    flat = pred_logits.reshape(B, N * C)

    scores64, labels64, rows64 = pl.pallas_call(
        functools.partial(_topk_kernel, num_classes=C),
        out_shape=[
            jax.ShapeDtypeStruct((B, _KPAD), jnp.float32),
            jax.ShapeDtypeStruct((B, _KPAD), jnp.int32),
            jax.ShapeDtypeStruct((B, _KPAD), jnp.int32),
        ],
        scratch_shapes=[pltpu.VMEM((B, N * C), jnp.float32)],
    )(flat)

    rows = rows64[:, :_NUM_SELECT]
    kp26 = pl.pallas_call(
        _gather_kernel,
        in_specs=[
            pl.BlockSpec(memory_space=pltpu.SMEM),
            pl.BlockSpec(memory_space=pltpu.SMEM),
            pl.BlockSpec(memory_space=pl.ANY),
        ],
        out_shape=jax.ShapeDtypeStruct((B, _NUM_SELECT, D), jnp.float32),
        scratch_shapes=[
            pltpu.VMEM((B, _NUM_SELECT, D), jnp.float32),
            pltpu.SemaphoreType.DMA,
        ],
    )(rows, target_sizes, pred_keypoints)

    scores = scores64[:, :_NUM_SELECT]
    labels = labels64[:, :_NUM_SELECT]
    kpr = kp26.reshape(B, _NUM_SELECT, _NBP, 2)
    kpr = jnp.concatenate([kpr, jnp.ones_like(kpr[..., :1])], axis=-1)
    return scores, labels, kpr.reshape(B, _NUM_SELECT, _NBP * 3)


# speedup vs baseline: 1.1782x; 1.0001x over previous
import functools

import jax
import jax.numpy as jnp
from jax.experimental import pallas as pl
from jax.experimental.pallas import tpu as pltpu

_NUM_SELECT = 60
_NBP = 13
_KPAD = 64
_PER_PASS = 4


def _topk_kernel(x_ref, scores_ref, labels_ref, rows_ref, xs_ref, *, num_classes):
    B, F = x_ref.shape
    xs_ref[...] = x_ref[...]
    col = jax.lax.broadcasted_iota(jnp.int32, (B, F), 1)
    lane = jax.lax.broadcasted_iota(jnp.int32, (B, _KPAD), 1)
    neg = jnp.float32(-jnp.inf)

    def body(i, carry):
        vals, idxs = carry
        x = xs_ref[...]
        for j in range(_PER_PASS):
            m = jnp.max(x, axis=1, keepdims=True)
            loc = jnp.min(jnp.where(x >= m, col, F), axis=1, keepdims=True)
            x = jnp.where(col == loc, neg, x)
            sel = lane == i * _PER_PASS + j
            vals = jnp.where(sel, m, vals)
            idxs = jnp.where(sel, loc, idxs)
        xs_ref[...] = x
        return vals, idxs

    vals = jnp.full((B, _KPAD), neg, jnp.float32)
    idxs = jnp.zeros((B, _KPAD), jnp.int32)
    vals, idxs = jax.lax.fori_loop(0, _NUM_SELECT // _PER_PASS, body, (vals, idxs))
    scores_ref[...] = jax.nn.sigmoid(vals)
    labels_ref[...] = idxs % num_classes
    rows_ref[...] = idxs // num_classes


def _gather_kernel(rows_ref, ts_ref, kp_ref, out_ref, scratch, sem):
    B, NS, D = out_ref.shape
    copies = []
    for b in range(B):
        for s in range(NS):
            c = pltpu.make_async_copy(
                kp_ref.at[b, rows_ref[b, s]],
                scratch.at[b, s],
                sem.at[(b * NS + s) % 8],
            )
            c.start()
            copies.append(c)
    for c in copies:
        c.wait()
    lane = jax.lax.broadcasted_iota(jnp.int32, (NS, D), 1)
    even = lane % 2 == 0
    for b in range(B):
        sx = ts_ref[b, 0]
        sy = ts_ref[b, 1]
        out_ref[b] = scratch[b] * jnp.where(even, sx, sy)


def kernel(pred_logits, pred_keypoints, target_sizes):
    B, N, C = pred_logits.shape
    D = pred_keypoints.shape[-1]
    flat = pred_logits.reshape(B, N * C)

    scores64, labels64, rows64 = pl.pallas_call(
        functools.partial(_topk_kernel, num_classes=C),
        out_shape=[
            jax.ShapeDtypeStruct((B, _KPAD), jnp.float32),
            jax.ShapeDtypeStruct((B, _KPAD), jnp.int32),
            jax.ShapeDtypeStruct((B, _KPAD), jnp.int32),
        ],
        scratch_shapes=[pltpu.VMEM((B, N * C), jnp.float32)],
    )(flat)

    rows = rows64[:, :_NUM_SELECT]
    kp26 = pl.pallas_call(
        _gather_kernel,
        in_specs=[
            pl.BlockSpec(memory_space=pltpu.SMEM),
            pl.BlockSpec(memory_space=pltpu.SMEM),
            pl.BlockSpec(memory_space=pl.ANY),
        ],
        out_shape=jax.ShapeDtypeStruct((B, _NUM_SELECT, D), jnp.float32),
        scratch_shapes=[
            pltpu.VMEM((B, _NUM_SELECT, D), jnp.float32),
            pltpu.SemaphoreType.DMA((8,)),
        ],
    )(rows, target_sizes, pred_keypoints)

    scores = scores64[:, :_NUM_SELECT]
    labels = labels64[:, :_NUM_SELECT]
    kpr = kp26.reshape(B, _NUM_SELECT, _NBP, 2)
    kpr = jnp.concatenate([kpr, jnp.ones_like(kpr[..., :1])], axis=-1)
    return scores, labels, kpr.reshape(B, _NUM_SELECT, _NBP * 3)
